# baseline (speedup 1.0000x reference)
import functools

import jax
import jax.numpy as jnp
from jax import lax
from jax.experimental import pallas as pl
from jax.experimental.pallas import tpu as pltpu
from jax.experimental.pallas import tpu_sc as plsc

_P_BLK = 4096
_F_BLK = 256


def _sc_gather_tri(verts_flat, faces_flat, f_pad):
    nw = 32
    fpw = f_pad // nw
    assert fpw % 16 == 0
    mesh = plsc.VectorSubcoreMesh(core_axis_name="c", subcore_axis_name="s")

    @functools.partial(
        pl.kernel,
        mesh=mesh,
        compiler_params=pltpu.CompilerParams(needs_layout_passes=False),
        out_type=jax.ShapeDtypeStruct((9, f_pad), jnp.float32),
        scratch_types=[
            pltpu.VMEM((verts_flat.shape[0],), jnp.float32),
            pltpu.VMEM((3 * fpw,), jnp.int32),
            pltpu.VMEM((9, fpw), jnp.float32),
        ],
    )
    def k(verts_hbm, faces_hbm, out_hbm, verts_v, faces_v, tri_v):
        wid = lax.axis_index("s") * 2 + lax.axis_index("c")
        base = wid * fpw
        pltpu.sync_copy(verts_hbm, verts_v)
        pltpu.sync_copy(faces_hbm.at[pl.ds(3 * base, 3 * fpw)], faces_v)
        iota3 = lax.iota(jnp.int32, 16) * 3
        for g in range(fpw // 16):
            for abc in range(3):
                vids = plsc.load_gather(faces_v, [iota3 + (48 * g + abc)])
                addr = vids * 3
                for c in range(3):
                    vals = plsc.load_gather(verts_v, [addr + c])
                    tri_v[3 * abc + c, pl.ds(16 * g, 16)] = vals
        pltpu.sync_copy(tri_v, out_hbm.at[:, pl.ds(base, fpw)])

    return k(verts_flat, faces_flat)


def _tc_body(pts_ref, tri_ref, dist_ref, assoc_ref, loss_ref):
    i = pl.program_id(0)
    j = pl.program_id(1)
    ni = pl.num_programs(0)
    nj = pl.num_programs(1)

    px = pts_ref[:, 0:1]
    py = pts_ref[:, 1:2]
    pz = pts_ref[:, 2:3]

    vax = tri_ref[0:1, :]
    vay = tri_ref[1:2, :]
    vaz = tri_ref[2:3, :]
    vbx = tri_ref[3:4, :]
    vby = tri_ref[4:5, :]
    vbz = tri_ref[5:6, :]
    vcx = tri_ref[6:7, :]
    vcy = tri_ref[7:8, :]
    vcz = tri_ref[8:9, :]

    abx = vbx - vax
    aby = vby - vay
    abz = vbz - vaz
    acx = vcx - vax
    acy = vcy - vay
    acz = vcz - vaz

    apx = px - vax
    apy = py - vay
    apz = pz - vaz
    d1 = abx * apx + aby * apy + abz * apz
    d2 = acx * apx + acy * apy + acz * apz

    bpx = px - vbx
    bpy = py - vby
    bpz = pz - vbz
    d3 = abx * bpx + aby * bpy + abz * bpz
    d4 = acx * bpx + acy * bpy + acz * bpz

    cvx = px - vcx
    cvy = py - vcy
    cvz = pz - vcz
    d5 = abx * cvx + aby * cvy + abz * cvz
    d6 = acx * cvx + acy * cvy + acz * cvz

    vva = d3 * d6 - d5 * d4
    vvb = d5 * d2 - d1 * d6
    vvc = d1 * d4 - d3 * d2

    denom = vva + vvb + vvc
    denom_s = jnp.where(jnp.abs(denom) < 1e-12, 1.0, denom)
    v = vvb / denom_s
    w = vvc / denom_s
    cpx = vax + v * abx + w * acx
    cpy = vay + v * aby + w * acy
    cpz = vaz + v * abz + w * acz

    d43 = d4 - d3
    d56 = d5 - d6
    tbc_den = d43 + d56
    tbc = d43 / jnp.where(jnp.abs(tbc_den) < 1e-12, 1.0, tbc_den)
    cond_bc = (vva <= 0) & (d43 >= 0) & (d56 >= 0)
    cpx = jnp.where(cond_bc, vbx + tbc * (vcx - vbx), cpx)
    cpy = jnp.where(cond_bc, vby + tbc * (vcy - vby), cpy)
    cpz = jnp.where(cond_bc, vbz + tbc * (vcz - vbz), cpz)

    tac_den = d2 - d6
    tac = d2 / jnp.where(jnp.abs(tac_den) < 1e-12, 1.0, tac_den)
    cond_ac = (vvb <= 0) & (d2 >= 0) & (d6 <= 0)
    cpx = jnp.where(cond_ac, vax + tac * acx, cpx)
    cpy = jnp.where(cond_ac, vay + tac * acy, cpy)
    cpz = jnp.where(cond_ac, vaz + tac * acz, cpz)

    tab_den = d1 - d3
    tab = d1 / jnp.where(jnp.abs(tab_den) < 1e-12, 1.0, tab_den)
    cond_ab = (vvc <= 0) & (d1 >= 0) & (d3 <= 0)
    cpx = jnp.where(cond_ab, vax + tab * abx, cpx)
    cpy = jnp.where(cond_ab, vay + tab * aby, cpy)
    cpz = jnp.where(cond_ab, vaz + tab * abz, cpz)

    cond_c = (d6 >= 0) & (d5 <= d6)
    cpx = jnp.where(cond_c, vcx, cpx)
    cpy = jnp.where(cond_c, vcy, cpy)
    cpz = jnp.where(cond_c, vcz, cpz)
    cond_b = (d3 >= 0) & (d4 <= d3)
    cpx = jnp.where(cond_b, vbx, cpx)
    cpy = jnp.where(cond_b, vby, cpy)
    cpz = jnp.where(cond_b, vbz, cpz)
    cond_a = (d1 <= 0) & (d2 <= 0)
    cpx = jnp.where(cond_a, vax, cpx)
    cpy = jnp.where(cond_a, vay, cpy)
    cpz = jnp.where(cond_a, vaz, cpz)

    dx = px - cpx
    dy = py - cpy
    dz = pz - cpz
    sq = dx * dx + dy * dy + dz * dz

    blockmin = jnp.min(sq, axis=1, keepdims=True)
    lane = jax.lax.broadcasted_iota(jnp.int32, sq.shape, 1)
    gidx = lane + j * _F_BLK
    cidx = jnp.min(
        jnp.where(sq == blockmin, gidx, jnp.int32(2**30)),
        axis=1,
        keepdims=True,
    )

    @pl.when(j == 0)
    def _():
        dist_ref[0] = blockmin
        assoc_ref[0] = cidx

    @pl.when(j > 0)
    def _():
        prev = dist_ref[0]
        pidx = assoc_ref[0]
        better = blockmin < prev
        dist_ref[0] = jnp.where(better, blockmin, prev)
        assoc_ref[0] = jnp.where(better, cidx, pidx)

    del i
    @pl.when(j == nj - 1)
    def _():
        npts = ni * _P_BLK
        s = jnp.sum(dist_ref[0])
        loss_ref[0] = jnp.full((1, 128), s / npts, jnp.float32)


def _sweep(pts, tri):
    p = pts.shape[0]
    f_pad = tri.shape[1]
    ni = p // _P_BLK
    nj = f_pad // _F_BLK
    dist, assoc, loss = pl.pallas_call(
        _tc_body,
        grid=(ni, nj),
        in_specs=[
            pl.BlockSpec((_P_BLK, 3), lambda i, j: (i, 0)),
            pl.BlockSpec((9, _F_BLK), lambda i, j: (0, j)),
        ],
        out_specs=[
            pl.BlockSpec((1, _P_BLK, 1), lambda i, j: (i, 0, 0)),
            pl.BlockSpec((1, _P_BLK, 1), lambda i, j: (i, 0, 0)),
            pl.BlockSpec((1, 1, 128), lambda i, j: (i, 0, 0)),
        ],
        out_shape=[
            jax.ShapeDtypeStruct((ni, _P_BLK, 1), jnp.float32),
            jax.ShapeDtypeStruct((ni, _P_BLK, 1), jnp.int32),
            jax.ShapeDtypeStruct((ni, 1, 128), jnp.float32),
        ],
        compiler_params=pltpu.CompilerParams(
            dimension_semantics=("parallel", "arbitrary")),
    )(pts, tri)
    return dist.reshape(p), assoc.reshape(p), jnp.sum(loss[:, 0, 0])


def kernel(verts, faces, points):
    v = verts.shape[0]
    f = faces.shape[0]
    f_dense = ((f + _F_BLK - 1) // _F_BLK) * _F_BLK
    f_sc = ((f_dense + 511) // 512) * 512

    verts_ext = jnp.concatenate(
        [verts, jnp.full((1, 3), 1e9, jnp.float32)], axis=0)
    nvf = 3 * (v + 1)
    nvf_pad = ((nvf + 7) // 8) * 8
    verts_flat = jnp.concatenate(
        [verts_ext.reshape(-1), jnp.zeros((nvf_pad - nvf,), jnp.float32)])
    fi = faces.astype(jnp.int32)
    fi = jnp.concatenate(
        [fi, jnp.full((f_sc - f, 3), v, jnp.int32)], axis=0)
    tri = _sc_gather_tri(verts_flat, fi.reshape(-1), f_sc)

    dist, assoc, loss = _sweep(points, tri[:, :f_dense])
    return (loss, dist, assoc)


# speedup vs baseline: 1.0004x; 1.0004x over previous
import functools

import jax
import jax.numpy as jnp
from jax import lax
from jax.experimental import pallas as pl
from jax.experimental.pallas import tpu as pltpu
from jax.experimental.pallas import tpu_sc as plsc

_P_BLK = 4096
_F_BLK = 256


def _sc_gather_tri(verts_flat, faces_flat, f_pad):
    nw = 32
    fpw = f_pad // nw
    assert fpw % 16 == 0
    mesh = plsc.VectorSubcoreMesh(core_axis_name="c", subcore_axis_name="s")

    @functools.partial(
        pl.kernel,
        mesh=mesh,
        compiler_params=pltpu.CompilerParams(needs_layout_passes=False),
        out_type=jax.ShapeDtypeStruct((9, f_pad), jnp.float32),
        scratch_types=[
            pltpu.VMEM((verts_flat.shape[0],), jnp.float32),
            pltpu.VMEM((3 * fpw,), jnp.int32),
            pltpu.VMEM((9, fpw), jnp.float32),
        ],
    )
    def k(verts_hbm, faces_hbm, out_hbm, verts_v, faces_v, tri_v):
        wid = lax.axis_index("s") * 2 + lax.axis_index("c")
        base = wid * fpw
        pltpu.sync_copy(verts_hbm, verts_v)
        pltpu.sync_copy(faces_hbm.at[pl.ds(3 * base, 3 * fpw)], faces_v)
        iota3 = lax.iota(jnp.int32, 16) * 3
        for g in range(fpw // 16):
            for abc in range(3):
                vids = plsc.load_gather(faces_v, [iota3 + (48 * g + abc)])
                addr = vids * 3
                for c in range(3):
                    vals = plsc.load_gather(verts_v, [addr + c])
                    tri_v[3 * abc + c, pl.ds(16 * g, 16)] = vals
        pltpu.sync_copy(tri_v, out_hbm.at[:, pl.ds(base, fpw)])

    return k(verts_flat, faces_flat)


def _tc_body(pts_ref, tri_ref, dist_ref, assoc_ref, loss_ref):
    i = pl.program_id(0)
    j = pl.program_id(1)
    ni = pl.num_programs(0)
    nj = pl.num_programs(1)

    px = pts_ref[:, 0:1]
    py = pts_ref[:, 1:2]
    pz = pts_ref[:, 2:3]

    vax = tri_ref[0:1, :]
    vay = tri_ref[1:2, :]
    vaz = tri_ref[2:3, :]
    vbx = tri_ref[3:4, :]
    vby = tri_ref[4:5, :]
    vbz = tri_ref[5:6, :]
    vcx = tri_ref[6:7, :]
    vcy = tri_ref[7:8, :]
    vcz = tri_ref[8:9, :]

    abx = vbx - vax
    aby = vby - vay
    abz = vbz - vaz
    acx = vcx - vax
    acy = vcy - vay
    acz = vcz - vaz

    apx = px - vax
    apy = py - vay
    apz = pz - vaz
    d1 = abx * apx + aby * apy + abz * apz
    d2 = acx * apx + acy * apy + acz * apz

    bpx = px - vbx
    bpy = py - vby
    bpz = pz - vbz
    d3 = abx * bpx + aby * bpy + abz * bpz
    d4 = acx * bpx + acy * bpy + acz * bpz

    cvx = px - vcx
    cvy = py - vcy
    cvz = pz - vcz
    d5 = abx * cvx + aby * cvy + abz * cvz
    d6 = acx * cvx + acy * cvy + acz * cvz

    vva = d3 * d6 - d5 * d4
    vvb = d5 * d2 - d1 * d6
    vvc = d1 * d4 - d3 * d2

    denom = vva + vvb + vvc
    denom_s = jnp.where(jnp.abs(denom) < 1e-12, 1.0, denom)
    v = vvb / denom_s
    w = vvc / denom_s
    cpx = vax + v * abx + w * acx
    cpy = vay + v * aby + w * acy
    cpz = vaz + v * abz + w * acz

    d43 = d4 - d3
    d56 = d5 - d6
    tbc_den = d43 + d56
    tbc = d43 / jnp.where(jnp.abs(tbc_den) < 1e-12, 1.0, tbc_den)
    cond_bc = (vva <= 0) & (d43 >= 0) & (d56 >= 0)
    cpx = jnp.where(cond_bc, vbx + tbc * (vcx - vbx), cpx)
    cpy = jnp.where(cond_bc, vby + tbc * (vcy - vby), cpy)
    cpz = jnp.where(cond_bc, vbz + tbc * (vcz - vbz), cpz)

    tac_den = d2 - d6
    tac = d2 / jnp.where(jnp.abs(tac_den) < 1e-12, 1.0, tac_den)
    cond_ac = (vvb <= 0) & (d2 >= 0) & (d6 <= 0)
    cpx = jnp.where(cond_ac, vax + tac * acx, cpx)
    cpy = jnp.where(cond_ac, vay + tac * acy, cpy)
    cpz = jnp.where(cond_ac, vaz + tac * acz, cpz)

    tab_den = d1 - d3
    tab = d1 / jnp.where(jnp.abs(tab_den) < 1e-12, 1.0, tab_den)
    cond_ab = (vvc <= 0) & (d1 >= 0) & (d3 <= 0)
    cpx = jnp.where(cond_ab, vax + tab * abx, cpx)
    cpy = jnp.where(cond_ab, vay + tab * aby, cpy)
    cpz = jnp.where(cond_ab, vaz + tab * abz, cpz)

    cond_c = (d6 >= 0) & (d5 <= d6)
    cpx = jnp.where(cond_c, vcx, cpx)
    cpy = jnp.where(cond_c, vcy, cpy)
    cpz = jnp.where(cond_c, vcz, cpz)
    cond_b = (d3 >= 0) & (d4 <= d3)
    cpx = jnp.where(cond_b, vbx, cpx)
    cpy = jnp.where(cond_b, vby, cpy)
    cpz = jnp.where(cond_b, vbz, cpz)
    cond_a = (d1 <= 0) & (d2 <= 0)
    cpx = jnp.where(cond_a, vax, cpx)
    cpy = jnp.where(cond_a, vay, cpy)
    cpz = jnp.where(cond_a, vaz, cpz)

    dx = px - cpx
    dy = py - cpy
    dz = pz - cpz
    sq = dx * dx + dy * dy + dz * dz

    blockmin = jnp.min(sq, axis=1, keepdims=True)
    lane = jax.lax.broadcasted_iota(jnp.int32, sq.shape, 1)
    gidx = lane + j * _F_BLK
    cidx = jnp.min(
        jnp.where(sq == blockmin, gidx, jnp.int32(2**30)),
        axis=1,
        keepdims=True,
    )

    @pl.when(j == 0)
    def _():
        dist_ref[0] = blockmin
        assoc_ref[0] = cidx

    @pl.when(j > 0)
    def _():
        prev = dist_ref[0]
        pidx = assoc_ref[0]
        better = blockmin < prev
        dist_ref[0] = jnp.where(better, blockmin, prev)
        assoc_ref[0] = jnp.where(better, cidx, pidx)

    @pl.when(j == nj - 1)
    def _():
        npts = ni * _P_BLK
        s = jnp.sum(dist_ref[0]).reshape(1, 1)
        prev = jnp.where(i == 0, jnp.zeros((1, 1), jnp.float32),
                         loss_ref[0:1, 0:1])
        tot = prev + s
        tot = jnp.where(i == ni - 1, tot / npts, tot)
        loss_ref[0:1, 0:1] = tot


def _sweep(pts, tri):
    p = pts.shape[0]
    f_pad = tri.shape[1]
    ni = p // _P_BLK
    nj = f_pad // _F_BLK
    dist, assoc, loss = pl.pallas_call(
        _tc_body,
        grid=(ni, nj),
        in_specs=[
            pl.BlockSpec((_P_BLK, 3), lambda i, j: (i, 0)),
            pl.BlockSpec((9, _F_BLK), lambda i, j: (0, j)),
        ],
        out_specs=[
            pl.BlockSpec((1, _P_BLK, 1), lambda i, j: (i, 0, 0)),
            pl.BlockSpec((1, _P_BLK, 1), lambda i, j: (i, 0, 0)),
            pl.BlockSpec((1, 1), lambda i, j: (0, 0)),
        ],
        out_shape=[
            jax.ShapeDtypeStruct((ni, _P_BLK, 1), jnp.float32),
            jax.ShapeDtypeStruct((ni, _P_BLK, 1), jnp.int32),
            jax.ShapeDtypeStruct((1, 1), jnp.float32),
        ],
    )(pts, tri)
    return dist.reshape(p), assoc.reshape(p), loss[0, 0]


def kernel(verts, faces, points):
    v = verts.shape[0]
    f = faces.shape[0]
    f_dense = ((f + _F_BLK - 1) // _F_BLK) * _F_BLK
    f_sc = ((f_dense + 511) // 512) * 512

    verts_ext = jnp.concatenate(
        [verts, jnp.full((1, 3), 1e9, jnp.float32)], axis=0)
    nvf = 3 * (v + 1)
    nvf_pad = ((nvf + 7) // 8) * 8
    verts_flat = jnp.concatenate(
        [verts_ext.reshape(-1), jnp.zeros((nvf_pad - nvf,), jnp.float32)])
    fi = faces.astype(jnp.int32)
    fi = jnp.concatenate(
        [fi, jnp.full((f_sc - f, 3), v, jnp.int32)], axis=0)
    tri = _sc_gather_tri(verts_flat, fi.reshape(-1), f_sc)

    dist, assoc, loss = _sweep(points, tri[:, :f_dense])
    return (loss, dist, assoc)
